# Initial kernel scaffold; baseline (speedup 1.0000x reference)
import functools

import jax
import jax.numpy as jnp
from jax import lax
from jax.experimental import pallas as pl
from jax.experimental.pallas import tpu as pltpu
from jax.experimental.pallas import tpu_sc as plsc

VOCAB = 32768
D = 64
CHUNK = 128
NBUF = 8
NW = 32


def _make_kernel(n_rows: int):
    rows_per_w = n_rows // NW
    chunks_per_w = rows_per_w // CHUNK
    groups = chunks_per_w // NBUF

    mesh = plsc.VectorSubcoreMesh(core_axis_name="c", subcore_axis_name="s")

    @functools.partial(
        pl.kernel,
        mesh=mesh,
        out_type=jax.ShapeDtypeStruct((n_rows, D), jnp.float32),
        scratch_types=[
            pltpu.VMEM((chunks_per_w, CHUNK), jnp.int32),
            pltpu.VMEM((NBUF, CHUNK, D), jnp.float32),
        ]
        + [pltpu.SemaphoreType.DMA] * NBUF
        + [pltpu.SemaphoreType.DMA] * NBUF,
    )
    def k(x_hbm, table_hbm, out_hbm, idx_v, rows_v, *sems):
        gsem = sems[:NBUF]
        ssem = sems[NBUF:]
        wid = lax.axis_index("s") * 2 + lax.axis_index("c")
        chunk0 = wid * chunks_per_w

        pltpu.sync_copy(x_hbm.at[pl.ds(chunk0, chunks_per_w)], idx_v)

        def start_gather(g, b):
            pltpu.make_async_copy(
                table_hbm.at[idx_v.at[g]], rows_v.at[b], gsem[b]
            ).start()

        def start_store(g, b):
            row = pl.multiple_of((chunk0 + g) * CHUNK, CHUNK)
            pltpu.make_async_copy(
                rows_v.at[b], out_hbm.at[pl.ds(row, CHUNK)], ssem[b]
            ).start()

        def wait_gather(b):
            pltpu.make_async_copy(
                table_hbm.at[idx_v.at[0]], rows_v.at[b], gsem[b]
            ).wait()

        def wait_store(b):
            row = pl.multiple_of(chunk0 * CHUNK, CHUNK)
            pltpu.make_async_copy(
                rows_v.at[b], out_hbm.at[pl.ds(row, CHUNK)], ssem[b]
            ).wait()

        def body(i, _):
            for b in range(NBUF):
                g = i * NBUF + b

                @pl.when(i > 0)
                def _():
                    wait_store(b)

                start_gather(g, b)
            for b in range(NBUF):
                g = i * NBUF + b
                wait_gather(b)
                start_store(g, b)
            return ()

        lax.fori_loop(0, groups, body, (), unroll=False)
        for b in range(NBUF):
            wait_store(b)

    return k


def kernel(x, table):
    b, s = x.shape
    n_rows = b * s
    xf = x.reshape(n_rows).astype(jnp.int32).reshape(n_rows // CHUNK, CHUNK)
    out = _make_kernel(n_rows)(xf, table)
    return out.reshape(b, s, D)


# speedup vs baseline: 4.4178x; 4.4178x over previous
import functools

import jax
import jax.numpy as jnp
from jax import lax
from jax.experimental import pallas as pl
from jax.experimental.pallas import tpu as pltpu
from jax.experimental.pallas import tpu_sc as plsc

VOCAB = 32768
D = 64
CHUNK = 128
NBUF = 8
NW = 32


def _make_kernel(n_rows: int):
    rows_per_w = n_rows // NW
    chunks_per_w = rows_per_w // CHUNK
    groups = chunks_per_w // NBUF

    mesh = plsc.VectorSubcoreMesh(core_axis_name="c", subcore_axis_name="s")

    @functools.partial(
        pl.kernel,
        mesh=mesh,
        out_type=jax.ShapeDtypeStruct((n_rows, D), jnp.float32),
        scratch_types=[
            pltpu.VMEM((chunks_per_w, CHUNK), jnp.int32),
            pltpu.VMEM((NBUF, CHUNK, D), jnp.float32),
        ]
        + [pltpu.SemaphoreType.DMA] * NBUF
        + [pltpu.SemaphoreType.DMA] * NBUF,
        compiler_params=pltpu.CompilerParams(use_tc_tiling_on_sc=False),
    )
    def k(x_hbm, table_hbm, out_hbm, idx_v, rows_v, *sems):
        gsem = sems[:NBUF]
        ssem = sems[NBUF:]
        wid = lax.axis_index("s") * 2 + lax.axis_index("c")
        chunk0 = wid * chunks_per_w

        pltpu.sync_copy(x_hbm.at[pl.ds(chunk0, chunks_per_w)], idx_v)

        def start_gather(g, b):
            pltpu.make_async_copy(
                table_hbm.at[idx_v.at[g]], rows_v.at[b], gsem[b]
            ).start()

        def start_store(g, b):
            row = pl.multiple_of((chunk0 + g) * CHUNK, CHUNK)
            pltpu.make_async_copy(
                rows_v.at[b], out_hbm.at[pl.ds(row, CHUNK)], ssem[b]
            ).start()

        def wait_gather(b):
            pltpu.make_async_copy(
                table_hbm.at[idx_v.at[0]], rows_v.at[b], gsem[b]
            ).wait()

        def wait_store(b):
            row = pl.multiple_of(chunk0 * CHUNK, CHUNK)
            pltpu.make_async_copy(
                rows_v.at[b], out_hbm.at[pl.ds(row, CHUNK)], ssem[b]
            ).wait()

        def body(i, _):
            for b in range(NBUF):
                g = i * NBUF + b

                @pl.when(i > 0)
                def _():
                    wait_store(b)

                start_gather(g, b)
            for b in range(NBUF):
                g = i * NBUF + b
                wait_gather(b)
                start_store(g, b)
            return ()

        lax.fori_loop(0, groups, body, (), unroll=False)
        for b in range(NBUF):
            wait_store(b)

    return k


def kernel(x, table):
    b, s = x.shape
    n_rows = b * s
    xf = x.reshape(n_rows).astype(jnp.int32).reshape(n_rows // CHUNK, CHUNK)
    out = _make_kernel(n_rows)(xf, table)
    return out.reshape(b, s, D)
